# Initial kernel scaffold; baseline (speedup 1.0000x reference)
import jax
import jax.numpy as jnp
from jax.experimental import pallas as pl


def kernel(x, emb):
    raise NotImplementedError("write your pallas kernel here")


# speedup vs baseline: 3.3146x; 3.3146x over previous
import functools

import jax
import jax.numpy as jnp
from jax import lax
from jax.experimental import pallas as pl
from jax.experimental.pallas import tpu as pltpu
from jax.experimental.pallas import tpu_sc as plsc


def _make_gather(B, V, D, n_cores, n_subcores, chunk):
    nw = n_cores * n_subcores
    assert B % nw == 0
    b_per_w = B // nw
    assert b_per_w % chunk == 0
    n_chunks = b_per_w // chunk
    mesh = plsc.VectorSubcoreMesh(core_axis_name="c", subcore_axis_name="s")

    @functools.partial(
        pl.kernel,
        mesh=mesh,
        out_type=jax.ShapeDtypeStruct((B, D), jnp.float32),
        scratch_types=[
            pltpu.VMEM((b_per_w,), jnp.int32),
            pltpu.VMEM((2, chunk, D), jnp.float32),
            pltpu.SemaphoreType.DMA,
            pltpu.SemaphoreType.DMA,
        ],
    )
    def gather_kernel(idx_hbm, emb_hbm, out_hbm, idx_v, rows_v, gsem, osem):
        wid = lax.axis_index("s") * n_cores + lax.axis_index("c")
        base = wid * b_per_w
        pltpu.sync_copy(idx_hbm.at[pl.ds(base, b_per_w)], idx_v)

        def start_gather(ch, buf):
            return pltpu.async_copy(
                emb_hbm.at[idx_v.at[pl.ds(ch * chunk, chunk)]],
                rows_v.at[buf],
                gsem,
            )

        def start_out(ch, buf):
            return pltpu.async_copy(
                rows_v.at[buf],
                out_hbm.at[pl.ds(base + ch * chunk, chunk)],
                osem,
            )

        gathers = [None] * n_chunks
        outs = [None] * n_chunks
        gathers[0] = start_gather(0, 0)
        for ch in range(n_chunks):
            buf = ch % 2
            gathers[ch].wait()
            if ch + 1 < n_chunks:
                if ch >= 1:
                    outs[ch - 1].wait()
                gathers[ch + 1] = start_gather(ch + 1, 1 - buf)
            outs[ch] = start_out(ch, buf)
        if n_chunks >= 2:
            outs[n_chunks - 2].wait()
        outs[n_chunks - 1].wait()

    return gather_kernel


def kernel(x, emb):
    B0, T = x.shape
    V, D = emb.shape
    B = B0 * T
    xf = x.reshape(B).astype(jnp.int32)
    info = plsc.get_sparse_core_info()
    fn = _make_gather(B, V, D, info.num_cores, info.num_subcores, chunk=400)
    out = fn(xf, emb)
    return out.reshape(B0, T, D)


# speedup vs baseline: 3.3612x; 1.0140x over previous
import functools

import jax
import jax.numpy as jnp
from jax import lax
from jax.experimental import pallas as pl
from jax.experimental.pallas import tpu as pltpu
from jax.experimental.pallas import tpu_sc as plsc


def _make_gather(B, V, D, n_cores, n_subcores, chunk, nbuf):
    nw = n_cores * n_subcores
    assert B % nw == 0
    b_per_w = B // nw
    assert b_per_w % chunk == 0
    n_chunks = b_per_w // chunk
    mesh = plsc.VectorSubcoreMesh(core_axis_name="c", subcore_axis_name="s")

    @functools.partial(
        pl.kernel,
        mesh=mesh,
        out_type=jax.ShapeDtypeStruct((B, D), jnp.float32),
        scratch_types=[
            pltpu.VMEM((b_per_w,), jnp.int32),
            pltpu.VMEM((nbuf, chunk, D), jnp.float32),
            pltpu.SemaphoreType.DMA,
            pltpu.SemaphoreType.DMA,
        ],
    )
    def gather_kernel(idx_hbm, emb_hbm, out_hbm, idx_v, rows_v, gsem, osem):
        wid = lax.axis_index("s") * n_cores + lax.axis_index("c")
        base = wid * b_per_w
        pltpu.sync_copy(idx_hbm.at[pl.ds(base, b_per_w)], idx_v)

        def start_gather(ch, buf):
            return pltpu.async_copy(
                emb_hbm.at[idx_v.at[pl.ds(ch * chunk, chunk)]],
                rows_v.at[buf],
                gsem,
            )

        def start_out(ch, buf):
            return pltpu.async_copy(
                rows_v.at[buf],
                out_hbm.at[pl.ds(base + ch * chunk, chunk)],
                osem,
            )

        gathers = [None] * n_chunks
        outs = [None] * n_chunks
        out_waited = [False] * n_chunks
        for b in range(min(nbuf - 1, n_chunks)):
            gathers[b] = start_gather(b, b % nbuf)
        for ch in range(n_chunks):
            buf = ch % nbuf
            gathers[ch].wait()
            outs[ch] = start_out(ch, buf)
            nxt = ch + nbuf - 1
            if nxt < n_chunks:
                prev = nxt - nbuf
                if prev >= 0:
                    outs[prev].wait()
                    out_waited[prev] = True
                gathers[nxt] = start_gather(nxt, nxt % nbuf)
        for ch in range(n_chunks):
            if not out_waited[ch]:
                outs[ch].wait()

    return gather_kernel


def kernel(x, emb):
    B0, T = x.shape
    V, D = emb.shape
    B = B0 * T
    xf = x.reshape(B).astype(jnp.int32)
    info = plsc.get_sparse_core_info()
    fn = _make_gather(B, V, D, info.num_cores, info.num_subcores, chunk=200, nbuf=4)
    out = fn(xf, emb)
    return out.reshape(B0, T, D)


# speedup vs baseline: 5.8656x; 1.7451x over previous
import functools

import jax
import jax.numpy as jnp
from jax import lax
from jax.experimental import pallas as pl
from jax.experimental.pallas import tpu as pltpu
from jax.experimental.pallas import tpu_sc as plsc


def _make_gather(B0, T, V, D, n_cores, n_subcores, rows_per_chunk, nbuf):
    nw = n_cores * n_subcores
    assert B0 % nw == 0
    rows_per_w = B0 // nw
    assert rows_per_w % rows_per_chunk == 0
    n_chunks = rows_per_w // rows_per_chunk
    chunk = rows_per_chunk * T
    b_per_w = rows_per_w * T
    mesh = plsc.VectorSubcoreMesh(core_axis_name="c", subcore_axis_name="s")

    @functools.partial(
        pl.kernel,
        mesh=mesh,
        out_type=jax.ShapeDtypeStruct((B0, T, D), jnp.float32),
        scratch_types=[
            pltpu.VMEM((b_per_w,), jnp.int32),
            pltpu.VMEM((nbuf, chunk, D), jnp.float32),
            pltpu.SemaphoreType.DMA,
            pltpu.SemaphoreType.DMA,
        ],
    )
    def gather_kernel(idx_hbm, emb_hbm, out_hbm, idx_v, rows_v, gsem, osem):
        wid = lax.axis_index("s") * n_cores + lax.axis_index("c")
        base = wid * b_per_w
        row_base = wid * rows_per_w
        pltpu.sync_copy(idx_hbm.at[pl.ds(base, b_per_w)], idx_v)

        def start_gather(ch, buf):
            return pltpu.async_copy(
                emb_hbm.at[idx_v.at[pl.ds(ch * chunk, chunk)]],
                rows_v.at[buf],
                gsem,
            )

        def start_out(ch, buf):
            cps = []
            for k in range(rows_per_chunk):
                cps.append(pltpu.async_copy(
                    rows_v.at[buf].at[pl.ds(k * T, T)],
                    out_hbm.at[row_base + ch * rows_per_chunk + k],
                    osem,
                ))
            return cps

        gathers = [None] * n_chunks
        outs = [None] * n_chunks
        out_waited = [False] * n_chunks

        def wait_out(ch):
            for cp in outs[ch]:
                cp.wait()
            out_waited[ch] = True

        for b in range(min(nbuf - 1, n_chunks)):
            gathers[b] = start_gather(b, b % nbuf)
        for ch in range(n_chunks):
            buf = ch % nbuf
            gathers[ch].wait()
            outs[ch] = start_out(ch, buf)
            nxt = ch + nbuf - 1
            if nxt < n_chunks:
                prev = nxt - nbuf
                if prev >= 0:
                    wait_out(prev)
                gathers[nxt] = start_gather(nxt, nxt % nbuf)
        for ch in range(n_chunks):
            if not out_waited[ch]:
                wait_out(ch)

    return gather_kernel


def kernel(x, emb):
    B0, T = x.shape
    V, D = emb.shape
    xf = x.reshape(B0 * T).astype(jnp.int32)
    info = plsc.get_sparse_core_info()
    fn = _make_gather(B0, T, V, D, info.num_cores, info.num_subcores,
                      rows_per_chunk=4, nbuf=4)
    return fn(xf, emb)


# speedup vs baseline: 5.8760x; 1.0018x over previous
import functools

import jax
import jax.numpy as jnp
from jax import lax
from jax.experimental import pallas as pl
from jax.experimental.pallas import tpu as pltpu
from jax.experimental.pallas import tpu_sc as plsc


def _make_gather(B0, T, V, D, n_cores, n_subcores, rows_per_chunk, nbuf):
    nw = n_cores * n_subcores
    assert B0 % nw == 0
    rows_per_w = B0 // nw
    assert rows_per_w % rows_per_chunk == 0
    n_chunks = rows_per_w // rows_per_chunk
    chunk = rows_per_chunk * T
    b_per_w = rows_per_w * T
    mesh = plsc.VectorSubcoreMesh(core_axis_name="c", subcore_axis_name="s")

    @functools.partial(
        pl.kernel,
        mesh=mesh,
        compiler_params=pltpu.CompilerParams(use_tc_tiling_on_sc=True),
        out_type=jax.ShapeDtypeStruct((B0, T, D), jnp.float32),
        scratch_types=[
            pltpu.VMEM((b_per_w,), jnp.int32),
            pltpu.VMEM((nbuf, chunk, D), jnp.float32),
            pltpu.SemaphoreType.DMA,
            pltpu.SemaphoreType.DMA,
        ],
    )
    def gather_kernel(idx_hbm, emb_hbm, out_hbm, idx_v, rows_v, gsem, osem):
        wid = lax.axis_index("s") * n_cores + lax.axis_index("c")
        base = wid * b_per_w
        row_base = wid * rows_per_w
        pltpu.sync_copy(idx_hbm.at[pl.ds(base, b_per_w)], idx_v)

        def start_gather(ch, buf):
            return pltpu.async_copy(
                emb_hbm.at[idx_v.at[pl.ds(ch * chunk, chunk)]],
                rows_v.at[buf],
                gsem,
            )

        def start_out(ch, buf):
            cps = []
            for k in range(rows_per_chunk):
                cps.append(pltpu.async_copy(
                    rows_v.at[buf].at[pl.ds(k * T, T)],
                    out_hbm.at[row_base + ch * rows_per_chunk + k],
                    osem,
                ))
            return cps

        gathers = [None] * n_chunks
        outs = [None] * n_chunks
        out_waited = [False] * n_chunks

        def wait_out(ch):
            for cp in outs[ch]:
                cp.wait()
            out_waited[ch] = True

        for b in range(min(nbuf - 1, n_chunks)):
            gathers[b] = start_gather(b, b % nbuf)
        for ch in range(n_chunks):
            buf = ch % nbuf
            gathers[ch].wait()
            outs[ch] = start_out(ch, buf)
            nxt = ch + nbuf - 1
            if nxt < n_chunks:
                prev = nxt - nbuf
                if prev >= 0:
                    wait_out(prev)
                gathers[nxt] = start_gather(nxt, nxt % nbuf)
        for ch in range(n_chunks):
            if not out_waited[ch]:
                wait_out(ch)

    return gather_kernel


def kernel(x, emb):
    B0, T = x.shape
    V, D = emb.shape
    xf = x.reshape(B0 * T).astype(jnp.int32)
    info = plsc.get_sparse_core_info()
    fn = _make_gather(B0, T, V, D, info.num_cores, info.num_subcores,
                      rows_per_chunk=4, nbuf=4)
    return fn(xf, emb)
